# baseline (speedup 1.0000x reference)
import functools

import jax
import jax.numpy as jnp
from jax import lax
from jax.experimental import pallas as pl
from jax.experimental.pallas import tpu as pltpu
from jax.experimental.pallas import tpu_sc as plsc

LR = 0.01
WEIGHT_DECAY = 0.0001

N_ROWS = 100000
HALF = N_ROWS // 2
D = 128
N_OCC = 51200
NUM = 50.0

NC = 2
NS = 16
L = 16

CHUNK = 5120
PASSES = 10
STRIPE = CHUNK // NS
KR = 80
KZ = 40
TILE_OCC = N_OCC // NS
KB = 64
NBMAX = TILE_OCC // KB + 1
TRASH = CHUNK
PROWS = CHUNK // 128
CROWS = PROWS + 8


def _make_table_kernel(lr_eff):
  mesh = plsc.VectorSubcoreMesh(
      core_axis_name="c", subcore_axis_name="s",
      num_cores=NC, num_subcores=NS)

  @functools.partial(
      pl.kernel,
      out_type=jax.ShapeDtypeStruct((N_ROWS, D), jnp.float32),
      mesh=mesh,
      compiler_params=pltpu.CompilerParams(needs_layout_passes=False),
      scratch_types=[
          pltpu.VMEM_SHARED((CHUNK + 8, D), jnp.float32),
          pltpu.VMEM_SHARED((CROWS, D), jnp.float32),
          pltpu.VMEM((TILE_OCC,), jnp.int32),
          pltpu.VMEM((NBMAX, KB), jnp.int32),
          pltpu.VMEM((NBMAX, KB), jnp.int32),
          pltpu.VMEM((KB,), jnp.int32),
          pltpu.VMEM((KB,), jnp.int32),
          pltpu.VMEM((KB, D), jnp.float32),
          pltpu.VMEM((CROWS, D), jnp.float32),
          pltpu.VMEM((CROWS,), jnp.int32),
          pltpu.VMEM((KR, D), jnp.float32),
          pltpu.VMEM((KR, D), jnp.float32),
          pltpu.VMEM((CROWS, D), jnp.float32),
          pltpu.VMEM((KZ, D), jnp.float32),
          pltpu.SemaphoreType.DMA,
      ],
  )
  def table_kernel(emb, grads, idx_hbm, out,
                   acc, cnt_p, idxbuf, pos_mat, lidx_mat, idx_stage,
                   lidx_stage, gbuf, cntloc, idn, ebuf, abuf, cbufp,
                   zbuf, dsem):
    c = lax.axis_index("c")
    s = lax.axis_index("s")
    iota = lax.iota(jnp.int32, L)
    e1 = jnp.float32(1.0 - WEIGHT_DECAY)
    limit = (c + 1) * HALF

    def init(i, _):
      zrow = jnp.zeros((L,), jnp.float32)
      for j in range(D // L):
        zbuf[i, pl.ds(j * L, L)] = zrow
      return 0
    lax.fori_loop(0, KZ, init, 0)

    def init1(j, _):
      idn[pl.ds(j * L, L)] = j * L + iota
      return 0
    lax.fori_loop(0, CROWS // L, init1, 0)

    pltpu.sync_copy(idx_hbm.at[pl.ds(s * TILE_OCC, TILE_OCC)], idxbuf)

    def pass_body(p, _):
      base = c * HALF + p * CHUNK

      def zloop(k, _):
        r0 = s * STRIPE + k * KZ
        pltpu.sync_copy(zbuf, acc.at[pl.ds(r0, KZ)])
        return 0
      lax.fori_loop(0, STRIPE // KZ, zloop, 0)

      @pl.when(s == 0)
      def _zc():
        pltpu.sync_copy(zbuf, cnt_p.at[pl.ds(0, KZ)])
        pltpu.sync_copy(zbuf.at[pl.ds(0, CROWS - KZ)],
                        cnt_p.at[pl.ds(KZ, CROWS - KZ)])

      def zc2(i, _):
        for j in range(D // L):
          cntloc[i, pl.ds(j * L, L)] = jnp.zeros((L,), jnp.float32)
        return 0
      lax.fori_loop(0, CROWS, zc2, 0)

      def pf(i, _):
        for j in range(KB // L):
          lidx_mat[i, pl.ds(j * L, L)] = jnp.full((L,), TRASH, jnp.int32)
          pos_mat[i, pl.ds(j * L, L)] = jnp.zeros((L,), jnp.int32)
        return 0
      lax.fori_loop(0, NBMAX, pf, 0)

      plsc.subcore_barrier()

      one_v = jnp.full((L,), 1.0, jnp.float32)

      def filt(j, n):
        v = idxbuf[pl.ds(j * L, L)]
        local = v - base
        m = (local >= 0) & (local < CHUNK) & (v < limit)
        mi = m.astype(jnp.int32)
        pc = plsc.cumsum(mi)
        dest = jnp.where(m, n + pc - 1, jnp.int32(NBMAX * KB - 1))
        drow = dest >> 6
        dcol = dest & (KB - 1)
        plsc.store_scatter(lidx_mat, [drow, dcol], local)
        posv = s * TILE_OCC + j * L + iota
        plsc.store_scatter(pos_mat, [drow, dcol], posv)
        pm = jnp.where(m, local, jnp.int32(CHUNK))
        plsc.addupdate_scatter(cntloc, [pm >> 7, pm & (D - 1)], one_v)
        return n + jnp.sum(mi)
      n = lax.fori_loop(0, TILE_OCC // L, filt, jnp.int32(0))
      pltpu.sync_copy(cntloc, cnt_p.at[idn], add=True)

      nb = (n + (KB - 1)) >> 6
      def batch(b, _):
        for j in range(KB // L):
          idx_stage[pl.ds(j * L, L)] = pos_mat[b, pl.ds(j * L, L)]
          lidx_stage[pl.ds(j * L, L)] = lidx_mat[b, pl.ds(j * L, L)]
        pltpu.async_copy(grads.at[idx_stage], gbuf, dsem).wait()
        pltpu.sync_copy(gbuf, acc.at[lidx_stage], add=True)
        return 0
      lax.fori_loop(0, nb, batch, 0)

      plsc.subcore_barrier()

      g0 = base + s * STRIPE
      pltpu.sync_copy(cnt_p, cbufp)

      def sub(k, _):
        @pl.when(g0 + k * KR < limit)
        def _do():
          r0 = s * STRIPE + k * KR
          pltpu.sync_copy(emb.at[pl.ds(g0 + k * KR, KR)], ebuf)
          pltpu.sync_copy(acc.at[pl.ds(r0, KR)], abuf)

          def row(q, _):
            for h in range(2):
              r = 2 * q + h
              rl = r0 + r
              rowv = jnp.full((L,), rl >> 7, jnp.int32)
              colv = jnp.full((L,), rl & (D - 1), jnp.int32)
              csv = plsc.load_gather(cbufp, [rowv, colv])
              m = csv > 0.0
              safe = jnp.maximum(csv, 1.0)
              fm = jnp.where(m, lr_eff / safe, 0.0)
              e1m = jnp.where(m, e1, 1.0)
              for j in range(D // L):
                e = ebuf[r, pl.ds(j * L, L)]
                a = abuf[r, pl.ds(j * L, L)]
                ebuf[r, pl.ds(j * L, L)] = e * e1m - a * fm
            return 0
          lax.fori_loop(0, KR // 2, row, 0)

          pltpu.sync_copy(ebuf, out.at[pl.ds(g0 + k * KR, KR)])
        return 0
      lax.fori_loop(0, STRIPE // KR, sub, 0)
      return 0

    lax.fori_loop(0, PASSES, pass_body, 0)

  return table_kernel


def kernel(user_emb, item_emb, user_grads, item_grads,
           returned_users, returned_items):
  gu = user_grads.reshape(N_OCC, D)
  gi = item_grads.reshape(N_OCC, D)
  iu = returned_users.reshape(N_OCC)
  ii = returned_items.reshape(N_OCC)

  new_user = _make_table_kernel(jnp.float32(LR))(user_emb, gu, iu)
  new_item = _make_table_kernel(jnp.float32(LR * NUM))(item_emb, gi, ii)
  return (new_user, new_item)


# speedup vs baseline: 1.0453x; 1.0453x over previous
import functools

import jax
import jax.numpy as jnp
from jax import lax
from jax.experimental import pallas as pl
from jax.experimental.pallas import tpu as pltpu
from jax.experimental.pallas import tpu_sc as plsc

LR = 0.01
WEIGHT_DECAY = 0.0001

N_ROWS = 100000
HALF = N_ROWS // 2
D = 128
N_OCC = 51200
NUM = 50.0

NC = 2
NS = 16
L = 16

CHUNK = 5120
PASSES = 10
STRIPE = CHUNK // NS
KR = 80
KZ = 40
TILE_OCC = N_OCC // NS
KB = 64
NBMAX = TILE_OCC // KB + 1
TRASH = CHUNK
PROWS = CHUNK // 128
CROWS = PROWS + 8


def _make_table_kernel(lr_eff):
  mesh = plsc.VectorSubcoreMesh(
      core_axis_name="c", subcore_axis_name="s",
      num_cores=NC, num_subcores=NS)

  @functools.partial(
      pl.kernel,
      out_type=jax.ShapeDtypeStruct((N_ROWS, D), jnp.float32),
      mesh=mesh,
      compiler_params=pltpu.CompilerParams(needs_layout_passes=False),
      scratch_types=[
          pltpu.VMEM_SHARED((CHUNK + 8, D), jnp.float32),
          pltpu.VMEM_SHARED((CROWS, D), jnp.float32),
          pltpu.VMEM((TILE_OCC,), jnp.int32),
          pltpu.VMEM((NBMAX, KB), jnp.int32),
          pltpu.VMEM((NBMAX, KB), jnp.int32),
          pltpu.VMEM((KB,), jnp.int32),
          pltpu.VMEM((KB,), jnp.int32),
          pltpu.VMEM((KB, D), jnp.float32),
          pltpu.VMEM((CROWS, D), jnp.float32),
          pltpu.VMEM((CROWS,), jnp.int32),
          pltpu.VMEM((KR, D), jnp.float32),
          pltpu.VMEM((KR, D), jnp.float32),
          pltpu.VMEM((CROWS, D), jnp.float32),
          pltpu.VMEM((KZ, D), jnp.float32),
          pltpu.SemaphoreType.DMA,
      ],
  )
  def table_kernel(emb, grads, idx_hbm, out,
                   acc, cnt_p, idxbuf, pos_mat, lidx_mat, idx_stage,
                   lidx_stage, gbuf, cntloc, idn, ebuf, abuf, cbufp,
                   zbuf, dsem):
    c = lax.axis_index("c")
    s = lax.axis_index("s")
    iota = lax.iota(jnp.int32, L)
    e1 = jnp.float32(1.0 - WEIGHT_DECAY)
    limit = (c + 1) * HALF

    def init(i, _):
      zrow = jnp.zeros((L,), jnp.float32)
      for j in range(D // L):
        zbuf[i, pl.ds(j * L, L)] = zrow
      return 0
    lax.fori_loop(0, KZ, init, 0)

    def init1(j, _):
      idn[pl.ds(j * L, L)] = j * L + iota
      return 0
    lax.fori_loop(0, CROWS // L, init1, 0)

    pltpu.sync_copy(idx_hbm.at[pl.ds(s * TILE_OCC, TILE_OCC)], idxbuf)

    def pass_body(p, _):
      base = c * HALF + p * CHUNK

      def zloop(k, _):
        r0 = s * STRIPE + k * KZ
        pltpu.sync_copy(zbuf, acc.at[pl.ds(r0, KZ)])
        return 0
      lax.fori_loop(0, STRIPE // KZ, zloop, 0)

      @pl.when(s == 0)
      def _zc():
        pltpu.sync_copy(zbuf, cnt_p.at[pl.ds(0, KZ)])
        pltpu.sync_copy(zbuf.at[pl.ds(0, CROWS - KZ)],
                        cnt_p.at[pl.ds(KZ, CROWS - KZ)])

      def zc2(i, _):
        for j in range(D // L):
          cntloc[i, pl.ds(j * L, L)] = jnp.zeros((L,), jnp.float32)
        return 0
      lax.fori_loop(0, CROWS, zc2, 0)

      def pf(i, _):
        for j in range(KB // L):
          lidx_mat[i, pl.ds(j * L, L)] = jnp.full((L,), TRASH, jnp.int32)
          pos_mat[i, pl.ds(j * L, L)] = jnp.zeros((L,), jnp.int32)
        return 0
      lax.fori_loop(0, NBMAX, pf, 0)

      plsc.subcore_barrier()

      def filt(j, n):
        v = idxbuf[pl.ds(j * L, L)]
        local = v - base
        m = (local >= 0) & (local < CHUNK) & (v < limit)
        mi = m.astype(jnp.int32)
        pc = plsc.cumsum(mi)
        dest = jnp.where(m, n + pc - 1, jnp.int32(NBMAX * KB - 1))
        drow = dest >> 6
        dcol = dest & (KB - 1)
        plsc.store_scatter(lidx_mat, [drow, dcol], local)
        posv = s * TILE_OCC + j * L + iota
        plsc.store_scatter(pos_mat, [drow, dcol], posv)
        return n + jnp.sum(mi)
      n = lax.fori_loop(0, TILE_OCC // L, filt, jnp.int32(0))

      nb = (n + (KB - 1)) >> 6
      one_v = jnp.full((L,), 1.0, jnp.float32)

      def batch(b, _):
        for j in range(KB // L):
          lv = lidx_mat[b, pl.ds(j * L, L)]
          idx_stage[pl.ds(j * L, L)] = pos_mat[b, pl.ds(j * L, L)]
          lidx_stage[pl.ds(j * L, L)] = lv
          plsc.addupdate_scatter(cntloc, [lv >> 7, lv & (D - 1)], one_v)
        pltpu.async_copy(grads.at[idx_stage], gbuf, dsem).wait()
        pltpu.sync_copy(gbuf, acc.at[lidx_stage], add=True)
        return 0
      lax.fori_loop(0, nb, batch, 0)
      pltpu.sync_copy(cntloc, cnt_p.at[idn], add=True)

      plsc.subcore_barrier()

      g0 = base + s * STRIPE
      pltpu.sync_copy(cnt_p, cbufp)

      def sub(k, _):
        @pl.when(g0 + k * KR < limit)
        def _do():
          r0 = s * STRIPE + k * KR
          pltpu.sync_copy(emb.at[pl.ds(g0 + k * KR, KR)], ebuf)
          pltpu.sync_copy(acc.at[pl.ds(r0, KR)], abuf)

          def row(q, _):
            for h in range(2):
              r = 2 * q + h
              rl = r0 + r
              rowv = jnp.full((L,), rl >> 7, jnp.int32)
              colv = jnp.full((L,), rl & (D - 1), jnp.int32)
              csv = plsc.load_gather(cbufp, [rowv, colv])
              m = csv > 0.0
              safe = jnp.maximum(csv, 1.0)
              fm = jnp.where(m, lr_eff / safe, 0.0)
              e1m = jnp.where(m, e1, 1.0)
              for j in range(D // L):
                e = ebuf[r, pl.ds(j * L, L)]
                a = abuf[r, pl.ds(j * L, L)]
                ebuf[r, pl.ds(j * L, L)] = e * e1m - a * fm
            return 0
          lax.fori_loop(0, KR // 2, row, 0)

          pltpu.sync_copy(ebuf, out.at[pl.ds(g0 + k * KR, KR)])
        return 0
      lax.fori_loop(0, STRIPE // KR, sub, 0)
      return 0

    lax.fori_loop(0, PASSES, pass_body, 0)

  return table_kernel


def kernel(user_emb, item_emb, user_grads, item_grads,
           returned_users, returned_items):
  gu = user_grads.reshape(N_OCC, D)
  gi = item_grads.reshape(N_OCC, D)
  iu = returned_users.reshape(N_OCC)
  ii = returned_items.reshape(N_OCC)

  new_user = _make_table_kernel(jnp.float32(LR))(user_emb, gu, iu)
  new_item = _make_table_kernel(jnp.float32(LR * NUM))(item_emb, gi, ii)
  return (new_user, new_item)


# speedup vs baseline: 1.0458x; 1.0005x over previous
import functools

import jax
import jax.numpy as jnp
from jax import lax
from jax.experimental import pallas as pl
from jax.experimental.pallas import tpu as pltpu
from jax.experimental.pallas import tpu_sc as plsc

LR = 0.01
WEIGHT_DECAY = 0.0001

N_ROWS = 100000
HALF = N_ROWS // 2
D = 128
N_OCC = 51200
NUM = 50.0

NC = 2
NS = 16
L = 16

CHUNK = 5120
PASSES = 10
STRIPE = CHUNK // NS
KR = 80
KZ = 40
TILE_OCC = N_OCC // NS
KB = 64
NBMAX = TILE_OCC // KB + 1
TRASH = CHUNK
PROWS = CHUNK // 128
CROWS = PROWS + 8


def _make_table_kernel(lr_eff):
  mesh = plsc.VectorSubcoreMesh(
      core_axis_name="c", subcore_axis_name="s",
      num_cores=NC, num_subcores=NS)

  @functools.partial(
      pl.kernel,
      out_type=jax.ShapeDtypeStruct((N_ROWS, D), jnp.float32),
      mesh=mesh,
      compiler_params=pltpu.CompilerParams(needs_layout_passes=False),
      scratch_types=[
          pltpu.VMEM_SHARED((CHUNK + 8, D), jnp.float32),
          pltpu.VMEM_SHARED((CROWS, D), jnp.float32),
          pltpu.VMEM((TILE_OCC,), jnp.int32),
          pltpu.VMEM((NBMAX, KB), jnp.int32),
          pltpu.VMEM((NBMAX, KB), jnp.int32),
          pltpu.VMEM((KB,), jnp.int32),
          pltpu.VMEM((KB,), jnp.int32),
          pltpu.VMEM((KB, D), jnp.float32),
          pltpu.VMEM((CROWS, D), jnp.float32),
          pltpu.VMEM((CROWS,), jnp.int32),
          pltpu.VMEM((KR, D), jnp.float32),
          pltpu.VMEM((KR, D), jnp.float32),
          pltpu.VMEM((CROWS, D), jnp.float32),
          pltpu.VMEM((KZ, D), jnp.float32),
          pltpu.SemaphoreType.DMA,
      ],
  )
  def table_kernel(emb, grads, idx_hbm, out,
                   acc, cnt_p, idxbuf, pos_mat, lidx_mat, idx_stage,
                   lidx_stage, gbuf, cntloc, idn, ebuf, abuf, cbufp,
                   zbuf, dsem):
    c = lax.axis_index("c")
    s = lax.axis_index("s")
    iota = lax.iota(jnp.int32, L)
    e1 = jnp.float32(1.0 - WEIGHT_DECAY)
    limit = (c + 1) * HALF

    def init(i, _):
      zrow = jnp.zeros((L,), jnp.float32)
      for j in range(D // L):
        zbuf[i, pl.ds(j * L, L)] = zrow
      return 0
    lax.fori_loop(0, KZ, init, 0)

    def init1(j, _):
      idn[pl.ds(j * L, L)] = j * L + iota
      return 0
    lax.fori_loop(0, CROWS // L, init1, 0)

    pltpu.sync_copy(idx_hbm.at[pl.ds(s * TILE_OCC, TILE_OCC)], idxbuf)

    def pass_body(p, _):
      base = c * HALF + p * CHUNK

      def zloop(k, _):
        r0 = s * STRIPE + k * KZ
        pltpu.sync_copy(zbuf, acc.at[pl.ds(r0, KZ)])
        return 0
      lax.fori_loop(0, STRIPE // KZ, zloop, 0)

      @pl.when(s == 0)
      def _zc():
        off = 0
        while off < CROWS:
          ln = min(KZ, CROWS - off)
          pltpu.sync_copy(zbuf.at[pl.ds(0, ln)], cnt_p.at[pl.ds(off, ln)])
          off += ln

      def zc2(i, _):
        for j in range(D // L):
          cntloc[i, pl.ds(j * L, L)] = jnp.zeros((L,), jnp.float32)
        return 0
      lax.fori_loop(0, CROWS, zc2, 0)

      def pf(i, _):
        for j in range(KB // L):
          lidx_mat[i, pl.ds(j * L, L)] = jnp.full((L,), TRASH, jnp.int32)
          pos_mat[i, pl.ds(j * L, L)] = jnp.zeros((L,), jnp.int32)
        return 0
      lax.fori_loop(0, NBMAX, pf, 0)

      plsc.subcore_barrier()

      def filt(j, n):
        v = idxbuf[pl.ds(j * L, L)]
        local = v - base
        m = (local >= 0) & (local < CHUNK) & (v < limit)
        mi = m.astype(jnp.int32)
        pc = plsc.cumsum(mi)
        dest = jnp.where(m, n + pc - 1, jnp.int32(NBMAX * KB - 1))
        drow = dest >> 6
        dcol = dest & (KB - 1)
        plsc.store_scatter(lidx_mat, [drow, dcol], local)
        posv = s * TILE_OCC + j * L + iota
        plsc.store_scatter(pos_mat, [drow, dcol], posv)
        return n + jnp.sum(mi)
      n = lax.fori_loop(0, TILE_OCC // L, filt, jnp.int32(0))

      nb = (n + (KB - 1)) >> 6
      one_v = jnp.full((L,), 1.0, jnp.float32)

      def batch(b, _):
        for j in range(KB // L):
          lv = lidx_mat[b, pl.ds(j * L, L)]
          idx_stage[pl.ds(j * L, L)] = pos_mat[b, pl.ds(j * L, L)]
          lidx_stage[pl.ds(j * L, L)] = lv
          plsc.addupdate_scatter(cntloc, [lv >> 7, lv & (D - 1)], one_v)
        pltpu.async_copy(grads.at[idx_stage], gbuf, dsem).wait()
        pltpu.sync_copy(gbuf, acc.at[lidx_stage], add=True)
        return 0
      lax.fori_loop(0, nb, batch, 0)
      pltpu.sync_copy(cntloc, cnt_p.at[idn], add=True)

      plsc.subcore_barrier()

      g0 = base + s * STRIPE
      pltpu.sync_copy(cnt_p, cbufp)

      def sub(k, _):
        @pl.when(g0 + k * KR < limit)
        def _do():
          r0 = s * STRIPE + k * KR
          pltpu.sync_copy(emb.at[pl.ds(g0 + k * KR, KR)], ebuf)
          pltpu.sync_copy(acc.at[pl.ds(r0, KR)], abuf)

          def row(q, _):
            for h in range(2):
              r = 2 * q + h
              rl = r0 + r
              rowv = jnp.full((L,), rl >> 7, jnp.int32)
              colv = jnp.full((L,), rl & (D - 1), jnp.int32)
              csv = plsc.load_gather(cbufp, [rowv, colv])
              m = csv > 0.0
              safe = jnp.maximum(csv, 1.0)
              fm = jnp.where(m, lr_eff / safe, 0.0)
              e1m = jnp.where(m, e1, 1.0)
              for j in range(D // L):
                e = ebuf[r, pl.ds(j * L, L)]
                a = abuf[r, pl.ds(j * L, L)]
                ebuf[r, pl.ds(j * L, L)] = e * e1m - a * fm
            return 0
          lax.fori_loop(0, KR // 2, row, 0)

          pltpu.sync_copy(ebuf, out.at[pl.ds(g0 + k * KR, KR)])
        return 0
      lax.fori_loop(0, STRIPE // KR, sub, 0)
      return 0

    lax.fori_loop(0, PASSES, pass_body, 0)

  return table_kernel


def kernel(user_emb, item_emb, user_grads, item_grads,
           returned_users, returned_items):
  gu = user_grads.reshape(N_OCC, D)
  gi = item_grads.reshape(N_OCC, D)
  iu = returned_users.reshape(N_OCC)
  ii = returned_items.reshape(N_OCC)

  new_user = _make_table_kernel(jnp.float32(LR))(user_emb, gu, iu)
  new_item = _make_table_kernel(jnp.float32(LR * NUM))(item_emb, gi, ii)
  return (new_user, new_item)
